# Initial kernel scaffold; baseline (speedup 1.0000x reference)
import jax
import jax.numpy as jnp
from jax.experimental import pallas as pl


def kernel(x, noise, enc_W1, enc_b1, enc_W2, enc_b2, gcn1_W, gcn1_b, gcn2_W, gcn2_b, dec_W1, dec_b1, dec_W2, dec_b2):
    raise NotImplementedError("write your pallas kernel here")


# speedup vs baseline: 3.6354x; 3.6354x over previous
import jax
import jax.numpy as jnp
import numpy as np
from jax.experimental import pallas as pl
from jax.experimental.pallas import tpu as pltpu

_N = 8


def _bf(v):
    return v.astype(jnp.bfloat16).astype(jnp.float32)


def _body(xT_ref, nT_ref, encw1_ref, encb2_ref, g1b_ref, db2_ref,
          encC_ref, encW2_ref, g1W_ref, W2T_ref, dW1T_ref,
          g2b_ref, db1_ref, dw2_ref,
          lat_ref, adj_ref, rec_ref):
    f32 = jnp.float32
    Bb = xT_ref.shape[-1]

    xv = _bf(xT_ref[...])
    encC = encC_ref[...]

    lat0 = jnp.zeros((_N, Bb), f32)
    lat1 = jnp.zeros((_N, Bb), f32)
    for j in range(64):
        hj = _bf(jnp.maximum(xv * encw1_ref[j] + encC[:, j:j + 1], 0.0))
        lat0 = lat0 + hj * encW2_ref[j, 0]
        lat1 = lat1 + hj * encW2_ref[j, 1]
    lat0 = lat0 + encb2_ref[0]
    lat1 = lat1 + encb2_ref[1]

    def norm_coord(lat, nz):
        c = lat - jnp.mean(lat, axis=0, keepdims=True)
        var = jnp.sum(c * c, axis=0, keepdims=True) * (1.0 / 7.0)
        std = jnp.sqrt(var) + 1e-8
        return c * (3.0 / std) + nz * 0.05

    p0 = norm_coord(lat0, nT_ref[0])
    p1 = norm_coord(lat1, nT_ref[1])
    lat_ref[0] = p0
    lat_ref[1] = p1

    ar = jax.lax.broadcasted_iota(jnp.int32, (_N, 1), 0)
    a = {}
    for i in range(_N):
        for j in range(i + 1, _N):
            pi0, pj0 = p0[i:i + 1], p0[j:j + 1]
            pi1, pj1 = p1[i:i + 1], p1[j:j + 1]
            mid0 = (pi0 + pj0) * 0.5
            mid1 = (pi1 + pj1) * 0.5
            r2 = (pi0 - mid0) ** 2 + (pi1 - mid1) ** 2
            d2 = (p0 - mid0) ** 2 + (p1 - mid1) ** 2
            mask = (ar != i) & (ar != j)
            violf = jnp.where((d2 < r2) & mask, 1.0, 0.0)
            a[(i, j)] = 1.0 - jnp.max(violf, axis=0, keepdims=True)

    zrow = jnp.zeros((1, Bb), f32)
    Acols = []
    for j in range(_N):
        col = jnp.concatenate(
            [zrow if i == j else a[(min(i, j), max(i, j))] for i in range(_N)],
            axis=0)
        adj_ref[j] = col
        Acols.append(col + jnp.where(ar == j, 1.0, 0.0))

    deg = Acols[0]
    for j in range(1, _N):
        deg = deg + Acols[j]
    dinv = 1.0 / jnp.sqrt(deg)
    ncols = [_bf((dinv * Acols[j]) * dinv[j:j + 1]) for j in range(_N)]

    p0r, p1r = _bf(p0), _bf(p1)
    x1 = []
    for f in range(32):
        xw = _bf(p0r * g1W_ref[0, f] + p1r * g1W_ref[1, f])
        msg = ncols[0] * xw[0:1]
        for j in range(1, _N):
            msg = msg + ncols[j] * xw[j:j + 1]
        x1.append(_bf(jnp.maximum(msg + g1b_ref[f], 0.0)))

    cstack = jnp.concatenate(
        [jnp.sum(ncols[j], axis=0, keepdims=True) for j in range(_N)],
        axis=0) * 0.125
    srows = [jnp.sum(x1[f] * cstack, axis=0, keepdims=True) for f in range(32)]
    S = jnp.concatenate(srows, axis=0)

    P = jnp.dot(W2T_ref[...], _bf(S), preferred_element_type=f32)
    pooled = _bf(P + g2b_ref[...])
    dh = jnp.dot(dW1T_ref[...], pooled, preferred_element_type=f32) + db1_ref[...]
    dh = _bf(jnp.maximum(dh, 0.0))
    rec = jnp.sum(dh * dw2_ref[...], axis=0, keepdims=True) + db2_ref[0]
    rec_ref[...] = rec


def kernel(x, noise, enc_W1, enc_b1, enc_W2, enc_b2, gcn1_W, gcn1_b,
           gcn2_W, gcn2_b, dec_W1, dec_b1, dec_W2, dec_b2):
    B = x.shape[0]
    Bb = 512 if B % 512 == 0 else B
    grid = (B // Bb,)
    f32 = jnp.float32

    def bf(v):
        return jax.lax.optimization_barrier(v.astype(jnp.bfloat16)).astype(f32)

    xT = x.T
    noiseT = jnp.transpose(noise, (2, 1, 0))
    idx = jnp.arange(_N, dtype=f32)
    enc_W1r = bf(enc_W1)
    encC = idx[:, None] * enc_W1r[2][None, :] + enc_b1[None, :]
    g2bcol = gcn2_b[:, None]
    db1col = dec_b1[:, None]
    dw2c = bf(dec_W2[:, 1])[:, None]
    db2 = dec_b2[1][None]

    smem = pl.BlockSpec(memory_space=pltpu.SMEM)

    def vspec(shape):
        nd = len(shape)
        return pl.BlockSpec(shape, lambda i: (0,) * nd)

    latT, adjT, rec = pl.pallas_call(
        _body,
        grid=grid,
        in_specs=[
            pl.BlockSpec((_N, Bb), lambda i: (0, i)),
            pl.BlockSpec((2, _N, Bb), lambda i: (0, 0, i)),
            smem, smem, smem, smem,
            vspec((_N, 64)), vspec((64, 2)), vspec((2, 32)),
            vspec((32, 32)), vspec((64, 32)),
            vspec((32, 1)), vspec((64, 1)), vspec((64, 1)),
        ],
        out_specs=[
            pl.BlockSpec((2, _N, Bb), lambda i: (0, 0, i)),
            pl.BlockSpec((_N, _N, Bb), lambda i: (0, 0, i)),
            pl.BlockSpec((1, Bb), lambda i: (0, i)),
        ],
        out_shape=[
            jax.ShapeDtypeStruct((2, _N, B), f32),
            jax.ShapeDtypeStruct((_N, _N, B), f32),
            jax.ShapeDtypeStruct((1, B), f32),
        ],
        compiler_params=pltpu.CompilerParams(
            dimension_semantics=("parallel",)),
    )(xT, noiseT, enc_W1r[1], enc_b2, gcn1_b, db2,
      encC, bf(enc_W2), bf(gcn1_W), bf(gcn2_W.T), bf(dec_W1.T),
      g2bcol, db1col, dw2c)

    latent = latT.transpose(2, 1, 0)
    adj = adjT.transpose(2, 1, 0) > 0.5
    recon = jnp.broadcast_to(rec.reshape(B, 1), (B, _N))
    return recon, latent, adj
